# baseline (speedup 1.0000x reference)
import functools

import jax
import jax.numpy as jnp
from jax import lax
from jax.experimental import pallas as pl
from jax.experimental.pallas import tpu as pltpu
from jax.experimental.pallas import tpu_sc as plsc

WORD_DIM = 64
H = 320
N_TOK = 1024 * 50
NW = 32

W_IDX_ROWS = 512
N_PAD = W_IDX_ROWS * 128
W_ROWS_PER_WORKER = W_IDX_ROWS // NW
W_CHUNK = 4
W_SUPER = W_ROWS_PER_WORKER // W_CHUNK
CHUNK_ROWS = W_CHUNK * 128


def _sc_word_gather(pair_idx, wtab_pairs):
    mesh = plsc.VectorSubcoreMesh(core_axis_name="c", subcore_axis_name="s")

    @functools.partial(
        pl.kernel,
        out_type=jax.ShapeDtypeStruct((N_PAD, 128), jnp.float32),
        mesh=mesh,
        scratch_types=[
            pltpu.VMEM((W_ROWS_PER_WORKER, 128), jnp.int32),
            pltpu.VMEM((CHUNK_ROWS, 128), jnp.float32),
            pltpu.SemaphoreType.DMA,
        ],
        compiler_params=pltpu.CompilerParams(use_tc_tiling_on_sc=False),
    )
    def gather_kernel(idx_hbm, tab_hbm, out_w, idx_v, rows_v, sem):
        wid = lax.axis_index("s") * 2 + lax.axis_index("c")
        base_row = wid * W_ROWS_PER_WORKER
        pltpu.sync_copy(idx_hbm.at[pl.ds(base_row, W_ROWS_PER_WORKER)], idx_v)
        for g in range(W_SUPER):
            for k in range(W_CHUNK):
                pltpu.async_copy(
                    tab_hbm.at[idx_v.at[g * W_CHUNK + k]],
                    rows_v.at[pl.ds(k * 128, 128)],
                    sem,
                )
            pltpu.make_async_copy(
                out_w.at[pl.ds(0, CHUNK_ROWS)], rows_v, sem
            ).wait()
            pltpu.sync_copy(
                rows_v,
                out_w.at[pl.ds((base_row + g * W_CHUNK) * 128, CHUNK_ROWS)],
            )

    return gather_kernel(pair_idx, wtab_pairs)


BLK = 1600
BB = BLK // 50


def _tc_body(wr_ref, widx_ref, cidx_ref, bigtab_ref,
             wc0, bt0, bg0, wc1, bt1, bg1, o_ref):
    parity = widx_ref[...] & 1
    wr = wr_ref[...]
    w64 = jnp.where(parity == 0, wr[:, :WORD_DIM], wr[:, WORD_DIM:])
    c = cidx_ref[...]
    c_tiled = pltpu.repeat(c, 128, axis=1)
    vcode = (lax.broadcasted_iota(jnp.int32, (1, 2048), 1) >> 4
             ).astype(jnp.int16)
    onehot = jnp.where(c_tiled == vcode,
                       jnp.bfloat16(1.0), jnp.bfloat16(0.0))
    cf = jnp.dot(onehot, bigtab_ref[...], preferred_element_type=jnp.float32)
    x = jnp.concatenate([w64, cf], axis=1)
    for wcat, bt, bg in ((wc0, bt0, bg0), (wc1, bt1, bg1)):
        xb = x.astype(jnp.bfloat16)
        y = jnp.dot(xb, wcat[...], preferred_element_type=jnp.float32)
        g = jax.nn.sigmoid(y[:, :H] + bg[...])
        t = jax.nn.relu(y[:, 384:384 + H] + bt[...])
        x = x + g * (t - x)
    o_ref[...] = x.reshape(BB, 50, H)


def _tc_stage(word_rows, widx_col, cidx2d, bigtab,
              wc0, bt0, bg0, wc1, bt1, bg1):
    grid = (N_TOK // BLK,)
    full = pl.BlockSpec((H, 768), lambda i: (0, 0))
    vec = pl.BlockSpec((1, H), lambda i: (0, 0))
    return pl.pallas_call(
        _tc_body,
        grid=grid,
        in_specs=[
            pl.BlockSpec((BLK, 128), lambda i: (i, 0)),
            pl.BlockSpec((BLK, 1), lambda i: (i, 0)),
            pl.BlockSpec((BLK, 16), lambda i: (i, 0)),
            pl.BlockSpec((2048, 256), lambda i: (0, 0)),
            full, vec, vec, full, vec, vec,
        ],
        out_specs=pl.BlockSpec((BB, 50, H), lambda i: (i, 0, 0)),
        out_shape=jax.ShapeDtypeStruct((1024, 50, H), jnp.float32),
    )(word_rows, widx_col, cidx2d, bigtab,
      wc0, bt0, bg0, wc1, bt1, bg1)


def kernel(w_idx, c_idx, word_table, char_table,
           Wt0, bt0, Wg0, bg0, Wt1, bt1, Wg1, bg1):
    B, L = w_idx.shape
    wflat = w_idx.reshape(N_TOK).astype(jnp.int32)
    pad_tail = jnp.arange(N_PAD - N_TOK, dtype=jnp.int32)
    pair_idx = jnp.concatenate([wflat >> 1, pad_tail]).reshape(W_IDX_ROWS, 128)
    wtab_pairs = word_table.reshape(word_table.shape[0] // 2, 128)

    word_rows = _sc_word_gather(pair_idx, wtab_pairs)

    bigtab = (char_table[:, None, None, :]
              * jnp.eye(16, dtype=jnp.float32)[None, :, :, None]
              ).reshape(2048, 256)

    bf = jnp.bfloat16

    def wcat(wg, wt):
        z = jnp.zeros((H, 64), jnp.float32)
        return jnp.concatenate([wg.T, z, wt.T, z], axis=1).astype(bf)

    out = _tc_stage(
        word_rows, wflat.reshape(N_TOK, 1),
        c_idx.reshape(N_TOK, 16).astype(jnp.int16),
        bigtab.astype(bf),
        wcat(Wg0, Wt0), bt0.reshape(1, H), bg0.reshape(1, H),
        wcat(Wg1, Wt1), bt1.reshape(1, H), bg1.reshape(1, H),
    )
    return out


# speedup vs baseline: 1.0097x; 1.0097x over previous
import functools

import jax
import jax.numpy as jnp
from jax import lax
from jax.experimental import pallas as pl
from jax.experimental.pallas import tpu as pltpu
from jax.experimental.pallas import tpu_sc as plsc

WORD_DIM = 64
H = 320
N_TOK = 1024 * 50
NW = 32

W_IDX_ROWS = 512
N_PAD = W_IDX_ROWS * 128
W_ROWS_PER_WORKER = W_IDX_ROWS // NW
W_CHUNK = 4
W_SUPER = W_ROWS_PER_WORKER // W_CHUNK
CHUNK_ROWS = W_CHUNK * 128


def _sc_word_gather(pair_idx, wtab_pairs):
    mesh = plsc.VectorSubcoreMesh(core_axis_name="c", subcore_axis_name="s")

    @functools.partial(
        pl.kernel,
        out_type=jax.ShapeDtypeStruct((N_PAD, 128), jnp.float32),
        mesh=mesh,
        scratch_types=[
            pltpu.VMEM((W_ROWS_PER_WORKER, 128), jnp.int32),
            pltpu.VMEM((CHUNK_ROWS, 128), jnp.float32),
            pltpu.SemaphoreType.DMA,
        ],
        compiler_params=pltpu.CompilerParams(use_tc_tiling_on_sc=False),
    )
    def gather_kernel(idx_hbm, tab_hbm, out_w, idx_v, rows_v, sem):
        wid = lax.axis_index("s") * 2 + lax.axis_index("c")
        base_row = wid * W_ROWS_PER_WORKER
        pltpu.sync_copy(idx_hbm.at[pl.ds(base_row, W_ROWS_PER_WORKER)], idx_v)
        for g in range(W_SUPER):
            for k in range(W_CHUNK):
                pltpu.async_copy(
                    tab_hbm.at[idx_v.at[g * W_CHUNK + k]],
                    rows_v.at[pl.ds(k * 128, 128)],
                    sem,
                )
            pltpu.make_async_copy(
                out_w.at[pl.ds(0, CHUNK_ROWS)], rows_v, sem
            ).wait()
            pltpu.sync_copy(
                rows_v,
                out_w.at[pl.ds((base_row + g * W_CHUNK) * 128, CHUNK_ROWS)],
            )

    return gather_kernel(pair_idx, wtab_pairs)


BLK = 3200
BB = BLK // 50


def _tc_body(wr_ref, widx_ref, cidx_ref, bigtab_ref,
             wc0, bt0, bg0, wc1, bt1, bg1, o_ref):
    parity = widx_ref[...] & 1
    wr = wr_ref[...]
    w64 = jnp.where(parity == 0, wr[:, :WORD_DIM], wr[:, WORD_DIM:])
    c = cidx_ref[...]
    c_tiled = pltpu.repeat(c, 128, axis=1)
    vcode = (lax.broadcasted_iota(jnp.int32, (1, 2048), 1) >> 4
             ).astype(jnp.int16)
    onehot = jnp.where(c_tiled == vcode,
                       jnp.bfloat16(1.0), jnp.bfloat16(0.0))
    cf = jnp.dot(onehot, bigtab_ref[...], preferred_element_type=jnp.float32)
    x = jnp.concatenate([w64, cf], axis=1)
    for wcat, bt, bg in ((wc0, bt0, bg0), (wc1, bt1, bg1)):
        xb = x.astype(jnp.bfloat16)
        y = jnp.dot(xb, wcat[...], preferred_element_type=jnp.float32)
        g = jax.nn.sigmoid(y[:, :H] + bg[...])
        t = jax.nn.relu(y[:, 384:384 + H] + bt[...])
        x = x + g * (t - x)
    o_ref[...] = x.reshape(BB, 50, H)


def _tc_stage(word_rows, widx_col, cidx2d, bigtab,
              wc0, bt0, bg0, wc1, bt1, bg1):
    grid = (N_TOK // BLK,)
    full = pl.BlockSpec((H, 768), lambda i: (0, 0))
    vec = pl.BlockSpec((1, H), lambda i: (0, 0))
    return pl.pallas_call(
        _tc_body,
        grid=grid,
        in_specs=[
            pl.BlockSpec((BLK, 128), lambda i: (i, 0)),
            pl.BlockSpec((BLK, 1), lambda i: (i, 0)),
            pl.BlockSpec((BLK, 16), lambda i: (i, 0)),
            pl.BlockSpec((2048, 256), lambda i: (0, 0)),
            full, vec, vec, full, vec, vec,
        ],
        out_specs=pl.BlockSpec((BB, 50, H), lambda i: (i, 0, 0)),
        out_shape=jax.ShapeDtypeStruct((1024, 50, H), jnp.float32),
    )(word_rows, widx_col, cidx2d, bigtab,
      wc0, bt0, bg0, wc1, bt1, bg1)


def kernel(w_idx, c_idx, word_table, char_table,
           Wt0, bt0, Wg0, bg0, Wt1, bt1, Wg1, bg1):
    B, L = w_idx.shape
    wflat = w_idx.reshape(N_TOK).astype(jnp.int32)
    pad_tail = jnp.arange(N_PAD - N_TOK, dtype=jnp.int32)
    pair_idx = jnp.concatenate([wflat >> 1, pad_tail]).reshape(W_IDX_ROWS, 128)
    wtab_pairs = word_table.reshape(word_table.shape[0] // 2, 128)

    word_rows = _sc_word_gather(pair_idx, wtab_pairs)

    bigtab = (char_table[:, None, None, :]
              * jnp.eye(16, dtype=jnp.float32)[None, :, :, None]
              ).reshape(2048, 256)

    bf = jnp.bfloat16

    def wcat(wg, wt):
        z = jnp.zeros((H, 64), jnp.float32)
        return jnp.concatenate([wg.T, z, wt.T, z], axis=1).astype(bf)

    out = _tc_stage(
        word_rows, wflat.reshape(N_TOK, 1),
        c_idx.reshape(N_TOK, 16).astype(jnp.int16),
        bigtab.astype(bf),
        wcat(Wg0, Wt0), bt0.reshape(1, H), bg0.reshape(1, H),
        wcat(Wg1, Wt1), bt1.reshape(1, H), bg1.reshape(1, H),
    )
    return out


# speedup vs baseline: 1.0355x; 1.0256x over previous
import functools

import jax
import jax.numpy as jnp
from jax import lax
from jax.experimental import pallas as pl
from jax.experimental.pallas import tpu as pltpu
from jax.experimental.pallas import tpu_sc as plsc

WORD_DIM = 64
H = 320
N_TOK = 1024 * 50
NW = 32

W_IDX_ROWS = 512
N_PAD = W_IDX_ROWS * 128
W_ROWS_PER_WORKER = W_IDX_ROWS // NW
W_CHUNK = 4
W_SUPER = W_ROWS_PER_WORKER // W_CHUNK
CHUNK_ROWS = W_CHUNK * 128


def _sc_word_gather(pair_idx, wtab_pairs):
    mesh = plsc.VectorSubcoreMesh(core_axis_name="c", subcore_axis_name="s")

    @functools.partial(
        pl.kernel,
        out_type=jax.ShapeDtypeStruct((N_PAD, 128), jnp.float32),
        mesh=mesh,
        scratch_types=[
            pltpu.VMEM((W_ROWS_PER_WORKER, 128), jnp.int32),
            pltpu.VMEM((CHUNK_ROWS, 128), jnp.float32),
            pltpu.SemaphoreType.DMA,
        ],
        compiler_params=pltpu.CompilerParams(use_tc_tiling_on_sc=False),
    )
    def gather_kernel(idx_hbm, tab_hbm, out_w, idx_v, rows_v, sem):
        wid = lax.axis_index("s") * 2 + lax.axis_index("c")
        base_row = wid * W_ROWS_PER_WORKER
        pltpu.sync_copy(idx_hbm.at[pl.ds(base_row, W_ROWS_PER_WORKER)], idx_v)
        for g in range(W_SUPER):
            for k in range(W_CHUNK):
                pltpu.async_copy(
                    tab_hbm.at[idx_v.at[g * W_CHUNK + k]],
                    rows_v.at[pl.ds(k * 128, 128)],
                    sem,
                )
            pltpu.make_async_copy(
                out_w.at[pl.ds(0, CHUNK_ROWS)], rows_v, sem
            ).wait()
            pltpu.sync_copy(
                rows_v,
                out_w.at[pl.ds((base_row + g * W_CHUNK) * 128, CHUNK_ROWS)],
            )

    return gather_kernel(pair_idx, wtab_pairs)


BLK = 3200
BB = BLK // 50


def _tc_body(wr_ref, cidx_ref, bigtab_ref,
             wc0, bt0, bg0, wc1, bt1, bg1, o_ref):
    ce = cidx_ref[...]
    parity = ce[:, 16:17]
    wr = wr_ref[...]
    w64 = jnp.where(parity == jnp.int16(0), wr[:, :WORD_DIM], wr[:, WORD_DIM:])
    c = ce[:, :16]
    c_tiled = pltpu.repeat(c, 128, axis=1)
    vcode = (lax.broadcasted_iota(jnp.int32, (1, 2048), 1) >> 4
             ).astype(jnp.int16)
    onehot = jnp.where(c_tiled == vcode,
                       jnp.bfloat16(1.0), jnp.bfloat16(0.0))
    cf = jnp.dot(onehot, bigtab_ref[...], preferred_element_type=jnp.float32)
    x = jnp.concatenate([w64, cf], axis=1)
    for wcat, bt, bg in ((wc0, bt0, bg0), (wc1, bt1, bg1)):
        xb = x.astype(jnp.bfloat16)
        y = jnp.dot(xb, wcat[...], preferred_element_type=jnp.float32)
        g = jax.nn.sigmoid(y[:, :H] + bg[...])
        t = jax.nn.relu(y[:, 384:384 + H] + bt[...])
        x = x + g * (t - x)
    o_ref[...] = x.reshape(BB, 50, H)


def _tc_stage(word_rows, cidx2d, bigtab,
              wc0, bt0, bg0, wc1, bt1, bg1):
    grid = (N_TOK // BLK,)
    full = pl.BlockSpec((H, 768), lambda i: (0, 0))
    vec = pl.BlockSpec((1, H), lambda i: (0, 0))
    return pl.pallas_call(
        _tc_body,
        grid=grid,
        in_specs=[
            pl.BlockSpec((BLK, 128), lambda i: (i, 0)),
            pl.BlockSpec((BLK, 17), lambda i: (i, 0)),
            pl.BlockSpec((2048, 256), lambda i: (0, 0)),
            full, vec, vec, full, vec, vec,
        ],
        out_specs=pl.BlockSpec((BB, 50, H), lambda i: (i, 0, 0)),
        out_shape=jax.ShapeDtypeStruct((1024, 50, H), jnp.float32),
    )(word_rows, cidx2d, bigtab,
      wc0, bt0, bg0, wc1, bt1, bg1)


def kernel(w_idx, c_idx, word_table, char_table,
           Wt0, bt0, Wg0, bg0, Wt1, bt1, Wg1, bg1):
    B, L = w_idx.shape
    wflat = w_idx.reshape(N_TOK).astype(jnp.int32)
    pad_tail = jnp.arange(N_PAD - N_TOK, dtype=jnp.int32)
    pair_idx = jnp.concatenate([wflat >> 1, pad_tail]).reshape(W_IDX_ROWS, 128)
    wtab_pairs = word_table.reshape(word_table.shape[0] // 2, 128)

    word_rows = _sc_word_gather(pair_idx, wtab_pairs)

    bigtab = (char_table[:, None, None, :]
              * jnp.eye(16, dtype=jnp.float32)[None, :, :, None]
              ).reshape(2048, 256)

    bf = jnp.bfloat16

    def wcat(wg, wt):
        z = jnp.zeros((H, 64), jnp.float32)
        return jnp.concatenate([wg.T, z, wt.T, z], axis=1).astype(bf)

    ci_ext = jnp.concatenate(
        [c_idx.reshape(N_TOK, 16).astype(jnp.int16),
         (wflat & 1).astype(jnp.int16).reshape(N_TOK, 1)], axis=1)
    out = _tc_stage(
        word_rows, ci_ext,
        bigtab.astype(bf),
        wcat(Wg0, Wt0), bt0.reshape(1, H), bg0.reshape(1, H),
        wcat(Wg1, Wt1), bt1.reshape(1, H), bg1.reshape(1, H),
    )
    return out
